# initial kernel scaffold (unmeasured)
import jax
import jax.numpy as jnp
from jax import lax
from jax.experimental import pallas as pl
from jax.experimental.pallas import tpu as pltpu

N_DEV = 8
SQ = 512
D = 1024
SKV = 2048
HQ_LOCAL = 8
HKV = 16
DH = 128
SCALE = 0.08838834764831843


def kernel(x, Wq, Wo, K_ext, V_ext):
    x2 = x.reshape(SQ, D)
    K2 = K_ext.reshape(SKV, HKV * DH)
    V2 = V_ext.reshape(SKV, HKV * DH)

    def body(x_ref, wq_ref, wo_ref, k_hbm, v_hbm, out_ref,
             k_vmem, v_vmem, attn_ref, comm_ref,
             local_sems, send_sems, recv_sems):
        my = lax.axis_index("i")
        right = lax.rem(my + 1, N_DEV)
        left = lax.rem(my + N_DEV - 1, N_DEV)

        col0 = my * 2 * DH
        k_copy = pltpu.make_async_copy(
            k_hbm.at[:, pl.ds(col0, 2 * DH)], k_vmem, local_sems.at[0])
        v_copy = pltpu.make_async_copy(
            v_hbm.at[:, pl.ds(col0, 2 * DH)], v_vmem, local_sems.at[1])
        k_copy.start()
        v_copy.start()

        barrier_sem = pltpu.get_barrier_semaphore()
        for nbr in [left, right]:
            pl.semaphore_signal(
                barrier_sem, inc=1,
                device_id=(nbr,), device_id_type=pl.DeviceIdType.MESH)
        pl.semaphore_wait(barrier_sem, 2)

        xb = x_ref[...].astype(jnp.bfloat16)
        wqb = wq_ref[...].astype(jnp.bfloat16)
        q = jnp.dot(xb, wqb, preferred_element_type=jnp.float32)
        qb = q.astype(jnp.bfloat16)

        k_copy.wait()
        v_copy.wait()
        kb = k_vmem[...].astype(jnp.bfloat16)
        vb = v_vmem[...].astype(jnp.bfloat16)

        for h in range(HQ_LOCAL):
            g = (h // 4) * DH
            qh = qb[:, h * DH:(h + 1) * DH]
            kh = kb[:, g:g + DH]
            s = lax.dot_general(
                qh, kh, (((1,), (1,)), ((), ())),
                preferred_element_type=jnp.float32) * SCALE
            m = jnp.max(s, axis=1, keepdims=True)
            p = jnp.exp(s - m)
            l = jnp.sum(p, axis=1, keepdims=True)
            oh = jnp.dot(p.astype(jnp.bfloat16), vb[:, g:g + DH],
                         preferred_element_type=jnp.float32) / l
            attn_ref[:, h * DH:(h + 1) * DH] = oh.astype(jnp.bfloat16)

        partial = jnp.dot(attn_ref[...], wo_ref[...].astype(jnp.bfloat16),
                          preferred_element_type=jnp.float32)
        out_ref[...] = partial
        comm_ref[0] = partial

        for hop in range(N_DEV - 1):
            s_slot = hop % 2
            r_slot = (hop + 1) % 2
            rdma = pltpu.make_async_remote_copy(
                src_ref=comm_ref.at[s_slot],
                dst_ref=comm_ref.at[r_slot],
                send_sem=send_sems.at[hop],
                recv_sem=recv_sems.at[hop],
                device_id=(right,),
                device_id_type=pl.DeviceIdType.MESH,
            )
            rdma.start()
            rdma.wait()
            out_ref[...] += comm_ref[r_slot]

    out = pl.pallas_call(
        body,
        out_shape=jax.ShapeDtypeStruct((SQ, D), jnp.float32),
        in_specs=[
            pl.BlockSpec(memory_space=pltpu.MemorySpace.VMEM),
            pl.BlockSpec(memory_space=pltpu.MemorySpace.VMEM),
            pl.BlockSpec(memory_space=pltpu.MemorySpace.VMEM),
            pl.BlockSpec(memory_space=pltpu.MemorySpace.ANY),
            pl.BlockSpec(memory_space=pltpu.MemorySpace.ANY),
        ],
        out_specs=pl.BlockSpec(memory_space=pltpu.MemorySpace.VMEM),
        scratch_shapes=[
            pltpu.VMEM((SKV, 2 * DH), jnp.float32),
            pltpu.VMEM((SKV, 2 * DH), jnp.float32),
            pltpu.VMEM((SQ, D), jnp.bfloat16),
            pltpu.VMEM((2, SQ, D), jnp.float32),
            pltpu.SemaphoreType.DMA((2,)),
            pltpu.SemaphoreType.DMA((N_DEV - 1,)),
            pltpu.SemaphoreType.DMA((N_DEV - 1,)),
        ],
        compiler_params=pltpu.CompilerParams(collective_id=0),
    )(x2, Wq, Wo, K2, V2)
    return out.reshape(1, SQ, D)


# baseline (device time: 220280 ns/iter reference)
import jax
import jax.numpy as jnp
from jax import lax
from jax.experimental import pallas as pl
from jax.experimental.pallas import tpu as pltpu

N_DEV = 8
SQ = 512
D = 1024
SKV = 2048
HQ_LOCAL = 8
HKV = 16
DH = 128
SCALE = 0.08838834764831843


def kernel(x, Wq, Wo, K_ext, V_ext):
    x2 = x.reshape(SQ, D)
    K2 = K_ext.reshape(SKV, HKV * DH)
    V2 = V_ext.reshape(SKV, HKV * DH)

    def body(x_ref, wq_ref, wo_ref, k_hbm, v_hbm, out_ref,
             k_vmem, v_vmem, attn_ref, comm_ref,
             local_sems, send_sems, recv_sems):
        my = lax.axis_index("i")
        right = lax.rem(my + 1, N_DEV)
        left = lax.rem(my + N_DEV - 1, N_DEV)

        col0 = my * 2 * DH
        k_copy = pltpu.make_async_copy(
            k_hbm.at[:, pl.ds(col0, 2 * DH)], k_vmem, local_sems.at[0])
        v_copy = pltpu.make_async_copy(
            v_hbm.at[:, pl.ds(col0, 2 * DH)], v_vmem, local_sems.at[1])
        k_copy.start()
        v_copy.start()

        barrier_sem = pltpu.get_barrier_semaphore()
        for nbr in [left, right]:
            pl.semaphore_signal(
                barrier_sem, inc=1,
                device_id=(nbr,), device_id_type=pl.DeviceIdType.MESH)
        pl.semaphore_wait(barrier_sem, 2)

        xb = x_ref[...].astype(jnp.bfloat16)
        wqb = wq_ref[...].astype(jnp.bfloat16)
        q = jnp.dot(xb, wqb, preferred_element_type=jnp.float32)
        qb = q.astype(jnp.bfloat16)

        k_copy.wait()
        v_copy.wait()
        kb = k_vmem[...].astype(jnp.bfloat16)
        vb = v_vmem[...].astype(jnp.bfloat16)

        for h in range(HQ_LOCAL):
            g = (h // 4) * DH
            qh = qb[:, h * DH:(h + 1) * DH]
            kh = kb[:, g:g + DH]
            s = lax.dot_general(
                qh, kh, (((1,), (1,)), ((), ())),
                preferred_element_type=jnp.float32) * SCALE
            m = jnp.max(s, axis=1, keepdims=True)
            p = jnp.exp(s - m)
            l = jnp.sum(p, axis=1, keepdims=True)
            oh = jnp.dot(p.astype(jnp.bfloat16), vb[:, g:g + DH],
                         preferred_element_type=jnp.float32) / l
            attn_ref[:, h * DH:(h + 1) * DH] = oh.astype(jnp.bfloat16)

        partial = jnp.dot(attn_ref[...], wo_ref[...].astype(jnp.bfloat16),
                          preferred_element_type=jnp.float32)
        out_ref[...] = partial
        comm_ref[0] = partial

        for hop in range(N_DEV - 1):
            s_slot = hop % 2
            r_slot = (hop + 1) % 2
            rdma = pltpu.make_async_remote_copy(
                src_ref=comm_ref.at[s_slot],
                dst_ref=comm_ref.at[r_slot],
                send_sem=send_sems.at[hop],
                recv_sem=recv_sems.at[hop],
                device_id=(right,),
                device_id_type=pl.DeviceIdType.MESH,
            )
            rdma.start()
            rdma.wait()
            out_ref[...] += comm_ref[r_slot]

    out = pl.pallas_call(
        body,
        out_shape=jax.ShapeDtypeStruct((SQ, D), jnp.float32),
        in_specs=[
            pl.BlockSpec(memory_space=pltpu.MemorySpace.VMEM),
            pl.BlockSpec(memory_space=pltpu.MemorySpace.VMEM),
            pl.BlockSpec(memory_space=pltpu.MemorySpace.VMEM),
            pl.BlockSpec(memory_space=pltpu.MemorySpace.HBM),
            pl.BlockSpec(memory_space=pltpu.MemorySpace.HBM),
        ],
        out_specs=pl.BlockSpec(memory_space=pltpu.MemorySpace.VMEM),
        scratch_shapes=[
            pltpu.VMEM((SKV, 2 * DH), jnp.float32),
            pltpu.VMEM((SKV, 2 * DH), jnp.float32),
            pltpu.VMEM((SQ, D), jnp.bfloat16),
            pltpu.VMEM((2, SQ, D), jnp.float32),
            pltpu.SemaphoreType.DMA((2,)),
            pltpu.SemaphoreType.DMA((N_DEV - 1,)),
            pltpu.SemaphoreType.DMA((N_DEV - 1,)),
        ],
        compiler_params=pltpu.CompilerParams(collective_id=0),
    )(x2, Wq, Wo, K2, V2)
    return out.reshape(1, SQ, D)


# device time: 108732 ns/iter; 2.0259x vs baseline; 2.0259x over previous
import jax
import jax.numpy as jnp
from jax import lax
from jax.experimental import pallas as pl
from jax.experimental.pallas import tpu as pltpu

N_DEV = 8
SQ = 512
D = 1024
SKV = 2048
HQ_LOCAL = 8
HKV = 16
DH = 128
CH = SQ // N_DEV
SCALE = 0.08838834764831843


def kernel(x, Wq, Wo, K_ext, V_ext):
    x2 = x.reshape(SQ, D)
    K2 = K_ext.reshape(SKV, HKV * DH)
    V2 = V_ext.reshape(SKV, HKV * DH)

    def body(x_ref, wq_ref, wo_ref, k_hbm, v_hbm, out_ref,
             kv_vmem, wq_bf, wo_bf, kv_bf, comm_ref,
             local_sems, rs_send_sems, rs_recv_sems,
             ag_send_sems, ag_recv_sems):
        my = lax.axis_index("i")
        right = lax.rem(my + 1, N_DEV)
        left = lax.rem(my + N_DEV - 1, N_DEV)

        col0 = my * 2 * DH
        k_copy = pltpu.make_async_copy(
            k_hbm.at[:, pl.ds(col0, 2 * DH)], kv_vmem.at[0], local_sems.at[0])
        v_copy = pltpu.make_async_copy(
            v_hbm.at[:, pl.ds(col0, 2 * DH)], kv_vmem.at[1], local_sems.at[1])
        k_copy.start()
        v_copy.start()

        barrier_sem = pltpu.get_barrier_semaphore()
        for nbr in [left, right]:
            pl.semaphore_signal(
                barrier_sem, inc=1,
                device_id=(nbr,), device_id_type=pl.DeviceIdType.MESH)
        pl.semaphore_wait(barrier_sem, 2)

        wq_bf[...] = wq_ref[...].astype(jnp.bfloat16)
        wo_bf[...] = wo_ref[...].astype(jnp.bfloat16)
        k_copy.wait()
        v_copy.wait()
        kv_bf[...] = kv_vmem[...].astype(jnp.bfloat16)

        def compute_chunk(idx):
            r = idx * CH
            xb = x_ref[pl.ds(r, CH), :].astype(jnp.bfloat16)
            q = jnp.dot(xb, wq_bf[...], preferred_element_type=jnp.float32)
            qb = q.astype(jnp.bfloat16)
            heads = []
            for h in range(HQ_LOCAL):
                g = (h // 4) * DH
                qh = qb[:, h * DH:(h + 1) * DH]
                kh = kv_bf[0, :, g:g + DH]
                s = lax.dot_general(
                    qh, kh, (((1,), (1,)), ((), ())),
                    preferred_element_type=jnp.float32) * SCALE
                m = jnp.max(s, axis=1, keepdims=True)
                p = jnp.exp(s - m)
                l = jnp.sum(p, axis=1, keepdims=True)
                oh = jnp.dot(p.astype(jnp.bfloat16), kv_bf[1, :, g:g + DH],
                             preferred_element_type=jnp.float32) / l
                heads.append(oh.astype(jnp.bfloat16))
            attn = jnp.concatenate(heads, axis=1)
            out_ref[pl.ds(r, CH), :] = jnp.dot(
                attn, wo_bf[...], preferred_element_type=jnp.float32)

        compute_chunk(my)
        for s in range(N_DEV - 1):
            send_idx = lax.rem(my - s + N_DEV, N_DEV)
            recv_idx = lax.rem(my - s - 1 + N_DEV, N_DEV)
            rdma = pltpu.make_async_remote_copy(
                src_ref=out_ref.at[pl.ds(send_idx * CH, CH), :],
                dst_ref=comm_ref.at[s],
                send_sem=rs_send_sems.at[s],
                recv_sem=rs_recv_sems.at[s],
                device_id=(right,),
                device_id_type=pl.DeviceIdType.MESH,
            )
            rdma.start()
            compute_chunk(recv_idx)
            rdma.wait()
            out_ref[pl.ds(recv_idx * CH, CH), :] += comm_ref[s]

        for s in range(N_DEV - 1):
            idx = lax.rem(my + 1 - s + N_DEV, N_DEV)
            rows = pl.ds(idx * CH, CH)
            rdma = pltpu.make_async_remote_copy(
                src_ref=out_ref.at[rows, :],
                dst_ref=out_ref.at[rows, :],
                send_sem=ag_send_sems.at[s],
                recv_sem=ag_recv_sems.at[s],
                device_id=(right,),
                device_id_type=pl.DeviceIdType.MESH,
            )
            rdma.start()
            rdma.wait()

    out = pl.pallas_call(
        body,
        out_shape=jax.ShapeDtypeStruct((SQ, D), jnp.float32),
        in_specs=[
            pl.BlockSpec(memory_space=pltpu.MemorySpace.VMEM),
            pl.BlockSpec(memory_space=pltpu.MemorySpace.VMEM),
            pl.BlockSpec(memory_space=pltpu.MemorySpace.VMEM),
            pl.BlockSpec(memory_space=pltpu.MemorySpace.HBM),
            pl.BlockSpec(memory_space=pltpu.MemorySpace.HBM),
        ],
        out_specs=pl.BlockSpec(memory_space=pltpu.MemorySpace.VMEM),
        scratch_shapes=[
            pltpu.VMEM((2, SKV, 2 * DH), jnp.float32),
            pltpu.VMEM((D, D), jnp.bfloat16),
            pltpu.VMEM((D, D), jnp.bfloat16),
            pltpu.VMEM((2, SKV, 2 * DH), jnp.bfloat16),
            pltpu.VMEM((N_DEV - 1, CH, D), jnp.float32),
            pltpu.SemaphoreType.DMA((2,)),
            pltpu.SemaphoreType.DMA((N_DEV - 1,)),
            pltpu.SemaphoreType.DMA((N_DEV - 1,)),
            pltpu.SemaphoreType.DMA((N_DEV - 1,)),
            pltpu.SemaphoreType.DMA((N_DEV - 1,)),
        ],
        compiler_params=pltpu.CompilerParams(collective_id=0),
    )(x2, Wq, Wo, K2, V2)
    return out.reshape(1, SQ, D)


# device time: 86357 ns/iter; 2.5508x vs baseline; 1.2591x over previous
import jax
import jax.numpy as jnp
from jax import lax
from jax.experimental import pallas as pl
from jax.experimental.pallas import tpu as pltpu

N_DEV = 8
SQ = 512
D = 1024
SKV = 2048
HQ_LOCAL = 8
HKV = 16
DH = 128
CH = SQ // N_DEV
SCALE = 0.08838834764831843


def kernel(x, Wq, Wo, K_ext, V_ext):

    def body(x_ref, wq_ref, wo_ref, k_hbm, v_hbm, out_ref,
             kv_vmem, wq_bf, wo_bf, kv_bf, comm_ref,
             local_sems, rs_send_sems, rs_recv_sems,
             ag_send_sems, ag_recv_sems):
        my = lax.axis_index("i")
        right = lax.rem(my + 1, N_DEV)
        left = lax.rem(my + N_DEV - 1, N_DEV)

        copies = []
        for t, src in enumerate((k_hbm, v_hbm)):
            for h in range(2):
                c = pltpu.make_async_copy(
                    src.at[0, :, 2 * my + h, :],
                    kv_vmem.at[t, h],
                    local_sems.at[2 * t + h])
                c.start()
                copies.append(c)

        barrier_sem = pltpu.get_barrier_semaphore()
        for nbr in [left, right]:
            pl.semaphore_signal(
                barrier_sem, inc=1,
                device_id=(nbr,), device_id_type=pl.DeviceIdType.MESH)
        pl.semaphore_wait(barrier_sem, 2)

        wq_bf[...] = wq_ref[...].astype(jnp.bfloat16)
        wo_bf[...] = wo_ref[...].astype(jnp.bfloat16)
        for c in copies:
            c.wait()
        kv_bf[...] = kv_vmem[...].astype(jnp.bfloat16)

        def compute_chunk(idx):
            r = idx * CH
            xb = x_ref[0, pl.ds(r, CH), :].astype(jnp.bfloat16)
            q = jnp.dot(xb, wq_bf[...], preferred_element_type=jnp.float32)
            qb = q.astype(jnp.bfloat16)
            heads = []
            for h in range(HQ_LOCAL):
                g = h // 4
                qh = qb[:, h * DH:(h + 1) * DH]
                kh = kv_bf[0, g]
                s = lax.dot_general(
                    qh, kh, (((1,), (1,)), ((), ())),
                    preferred_element_type=jnp.float32) * SCALE
                m = jnp.max(s, axis=1, keepdims=True)
                p = jnp.exp(s - m)
                l = jnp.sum(p, axis=1, keepdims=True)
                oh = jnp.dot(p.astype(jnp.bfloat16), kv_bf[1, g],
                             preferred_element_type=jnp.float32) / l
                heads.append(oh.astype(jnp.bfloat16))
            attn = jnp.concatenate(heads, axis=1)
            out_ref[0, pl.ds(r, CH), :] = jnp.dot(
                attn, wo_bf[...], preferred_element_type=jnp.float32)

        compute_chunk(my)
        for s in range(N_DEV - 1):
            send_idx = lax.rem(my - s + N_DEV, N_DEV)
            recv_idx = lax.rem(my - s - 1 + N_DEV, N_DEV)
            rdma = pltpu.make_async_remote_copy(
                src_ref=out_ref.at[0, pl.ds(send_idx * CH, CH), :],
                dst_ref=comm_ref.at[s],
                send_sem=rs_send_sems.at[s],
                recv_sem=rs_recv_sems.at[s],
                device_id=(right,),
                device_id_type=pl.DeviceIdType.MESH,
            )
            rdma.start()
            compute_chunk(recv_idx)
            rdma.wait()
            out_ref[0, pl.ds(recv_idx * CH, CH), :] += comm_ref[s]

        for s in range(N_DEV - 1):
            idx = lax.rem(my + 1 - s + N_DEV, N_DEV)
            rows = pl.ds(idx * CH, CH)
            rdma = pltpu.make_async_remote_copy(
                src_ref=out_ref.at[0, rows, :],
                dst_ref=out_ref.at[0, rows, :],
                send_sem=ag_send_sems.at[s],
                recv_sem=ag_recv_sems.at[s],
                device_id=(right,),
                device_id_type=pl.DeviceIdType.MESH,
            )
            rdma.start()
            rdma.wait()

    return pl.pallas_call(
        body,
        out_shape=jax.ShapeDtypeStruct((1, SQ, D), jnp.float32),
        in_specs=[
            pl.BlockSpec(memory_space=pltpu.MemorySpace.VMEM),
            pl.BlockSpec(memory_space=pltpu.MemorySpace.VMEM),
            pl.BlockSpec(memory_space=pltpu.MemorySpace.VMEM),
            pl.BlockSpec(memory_space=pltpu.MemorySpace.HBM),
            pl.BlockSpec(memory_space=pltpu.MemorySpace.HBM),
        ],
        out_specs=pl.BlockSpec(memory_space=pltpu.MemorySpace.VMEM),
        scratch_shapes=[
            pltpu.VMEM((2, 2, SKV, DH), jnp.float32),
            pltpu.VMEM((D, D), jnp.bfloat16),
            pltpu.VMEM((D, D), jnp.bfloat16),
            pltpu.VMEM((2, 2, SKV, DH), jnp.bfloat16),
            pltpu.VMEM((N_DEV - 1, CH, D), jnp.float32),
            pltpu.SemaphoreType.DMA((4,)),
            pltpu.SemaphoreType.DMA((N_DEV - 1,)),
            pltpu.SemaphoreType.DMA((N_DEV - 1,)),
            pltpu.SemaphoreType.DMA((N_DEV - 1,)),
            pltpu.SemaphoreType.DMA((N_DEV - 1,)),
        ],
        compiler_params=pltpu.CompilerParams(collective_id=0),
    )(x, Wq, Wo, K_ext, V_ext)


# device time: 74900 ns/iter; 2.9410x vs baseline; 1.1530x over previous
import jax
import jax.numpy as jnp
from jax import lax
from jax.experimental import pallas as pl
from jax.experimental.pallas import tpu as pltpu

N_DEV = 8
SQ = 512
D = 1024
SKV = 2048
HQ_LOCAL = 8
HKV = 16
DH = 128
CH = SQ // N_DEV
SCALE = 0.08838834764831843


def kernel(x, Wq, Wo, K_ext, V_ext):

    def body(x_ref, wq_ref, wo_ref, k_hbm, v_hbm, out_ref,
             kv_vmem, wq_bf, wo_bf, kv_bf, comm_ref,
             local_sems, rs_send_sems, rs_recv_sems,
             ag_send_sems, ag_recv_sems, agl_send_sems, agl_recv_sems):
        my = lax.axis_index("i")
        right = lax.rem(my + 1, N_DEV)
        left = lax.rem(my + N_DEV - 1, N_DEV)

        copies = []
        for t, src in enumerate((k_hbm, v_hbm)):
            for h in range(2):
                c = pltpu.make_async_copy(
                    src.at[0, :, 2 * my + h, :],
                    kv_vmem.at[t, h],
                    local_sems.at[2 * t + h])
                c.start()
                copies.append(c)

        barrier_sem = pltpu.get_barrier_semaphore()
        for nbr in [left, right]:
            pl.semaphore_signal(
                barrier_sem, inc=1,
                device_id=(nbr,), device_id_type=pl.DeviceIdType.MESH)
        pl.semaphore_wait(barrier_sem, 2)

        wq_bf[...] = wq_ref[...].astype(jnp.bfloat16)
        wo_bf[...] = wo_ref[...].astype(jnp.bfloat16)
        for c in copies:
            c.wait()
        kv_bf[...] = kv_vmem[...].astype(jnp.bfloat16)

        def compute_chunk(idx):
            r = idx * CH
            xb = x_ref[0, pl.ds(r, CH), :].astype(jnp.bfloat16)
            q = jnp.dot(xb, wq_bf[...], preferred_element_type=jnp.float32)
            qb = q.astype(jnp.bfloat16)
            heads = []
            for h in range(HQ_LOCAL):
                g = h // 4
                qh = qb[:, h * DH:(h + 1) * DH]
                kh = kv_bf[0, g]
                s = lax.dot_general(
                    qh, kh, (((1,), (1,)), ((), ())),
                    preferred_element_type=jnp.float32) * SCALE
                m = jnp.max(s, axis=1, keepdims=True)
                p = jnp.exp(s - m)
                l = jnp.sum(p, axis=1, keepdims=True)
                oh = jnp.dot(p.astype(jnp.bfloat16), kv_bf[1, g],
                             preferred_element_type=jnp.float32) / l
                heads.append(oh.astype(jnp.bfloat16))
            attn = jnp.concatenate(heads, axis=1)
            out_ref[0, pl.ds(r, CH), :] = jnp.dot(
                attn, wo_bf[...], preferred_element_type=jnp.float32)

        compute_chunk(my)
        for s in range(N_DEV - 1):
            send_idx = lax.rem(my - s + N_DEV, N_DEV)
            recv_idx = lax.rem(my - s - 1 + N_DEV, N_DEV)
            rdma = pltpu.make_async_remote_copy(
                src_ref=out_ref.at[0, pl.ds(send_idx * CH, CH), :],
                dst_ref=comm_ref.at[s],
                send_sem=rs_send_sems.at[s],
                recv_sem=rs_recv_sems.at[s],
                device_id=(right,),
                device_id_type=pl.DeviceIdType.MESH,
            )
            rdma.start()
            compute_chunk(recv_idx)
            rdma.wait()
            out_ref[0, pl.ds(recv_idx * CH, CH), :] += comm_ref[s]

        for s in range(4):
            r_idx = lax.rem(my + 1 - s + N_DEV, N_DEV)
            rows_r = pl.ds(r_idx * CH, CH)
            rdma_r = pltpu.make_async_remote_copy(
                src_ref=out_ref.at[0, rows_r, :],
                dst_ref=out_ref.at[0, rows_r, :],
                send_sem=ag_send_sems.at[s],
                recv_sem=ag_recv_sems.at[s],
                device_id=(right,),
                device_id_type=pl.DeviceIdType.MESH,
            )
            rdma_r.start()
            if s < 3:
                l_idx = lax.rem(my + 1 + s, N_DEV)
                rows_l = pl.ds(l_idx * CH, CH)
                rdma_l = pltpu.make_async_remote_copy(
                    src_ref=out_ref.at[0, rows_l, :],
                    dst_ref=out_ref.at[0, rows_l, :],
                    send_sem=agl_send_sems.at[s],
                    recv_sem=agl_recv_sems.at[s],
                    device_id=(left,),
                    device_id_type=pl.DeviceIdType.MESH,
                )
                rdma_l.start()
                rdma_l.wait()
            rdma_r.wait()

    return pl.pallas_call(
        body,
        out_shape=jax.ShapeDtypeStruct((1, SQ, D), jnp.float32),
        in_specs=[
            pl.BlockSpec(memory_space=pltpu.MemorySpace.VMEM),
            pl.BlockSpec(memory_space=pltpu.MemorySpace.VMEM),
            pl.BlockSpec(memory_space=pltpu.MemorySpace.VMEM),
            pl.BlockSpec(memory_space=pltpu.MemorySpace.HBM),
            pl.BlockSpec(memory_space=pltpu.MemorySpace.HBM),
        ],
        out_specs=pl.BlockSpec(memory_space=pltpu.MemorySpace.VMEM),
        scratch_shapes=[
            pltpu.VMEM((2, 2, SKV, DH), jnp.float32),
            pltpu.VMEM((D, D), jnp.bfloat16),
            pltpu.VMEM((D, D), jnp.bfloat16),
            pltpu.VMEM((2, 2, SKV, DH), jnp.bfloat16),
            pltpu.VMEM((N_DEV - 1, CH, D), jnp.float32),
            pltpu.SemaphoreType.DMA((4,)),
            pltpu.SemaphoreType.DMA((N_DEV - 1,)),
            pltpu.SemaphoreType.DMA((N_DEV - 1,)),
            pltpu.SemaphoreType.DMA((4,)),
            pltpu.SemaphoreType.DMA((4,)),
            pltpu.SemaphoreType.DMA((3,)),
            pltpu.SemaphoreType.DMA((3,)),
        ],
        compiler_params=pltpu.CompilerParams(collective_id=0),
    )(x, Wq, Wo, K_ext, V_ext)


# device time: 39193 ns/iter; 5.6204x vs baseline; 1.9111x over previous
import jax
import jax.numpy as jnp
from jax import lax
from jax.experimental import pallas as pl
from jax.experimental.pallas import tpu as pltpu

N_DEV = 8
SQ = 512
D = 1024
SKV = 2048
HQ_LOCAL = 8
HKV = 16
DH = 128
CH = SQ // N_DEV
SCALE = 0.08838834764831843


def kernel(x, Wq, Wo, K_ext, V_ext):

    def body(x_ref, wq_ref, wo_ref, k_hbm, v_hbm, out_ref,
             kv_vmem, wq_bf, wo_bf, kv_bf, q_all,
             rland, rstage, lland, lstage, ag_own, ag_land,
             local_sems, r_send_sems, r_recv_sems, l_send_sems, l_recv_sems,
             ag_send_sems, ag_recv_sems):
        my = lax.axis_index("i")
        right = lax.rem(my + 1, N_DEV)
        left = lax.rem(my + N_DEV - 1, N_DEV)

        def chunk_rows(c):
            return pl.ds(lax.rem(c + 2 * N_DEV, N_DEV) * CH, CH)

        copies = []
        for t, src in enumerate((k_hbm, v_hbm)):
            for h in range(2):
                c = pltpu.make_async_copy(
                    src.at[0, :, 2 * my + h, :],
                    kv_vmem.at[t, h],
                    local_sems.at[2 * t + h])
                c.start()
                copies.append(c)

        barrier_sem = pltpu.get_barrier_semaphore()
        for nbr in [left, right]:
            pl.semaphore_signal(
                barrier_sem, inc=1,
                device_id=(nbr,), device_id_type=pl.DeviceIdType.MESH)
        pl.semaphore_wait(barrier_sem, 2)

        wq_bf[...] = wq_ref[...].astype(jnp.bfloat16)
        wo_bf[...] = wo_ref[...].astype(jnp.bfloat16)
        for c in copies:
            c.wait()
        kv_bf[...] = kv_vmem[...].astype(jnp.bfloat16)

        q_full = jnp.dot(x_ref[0].astype(jnp.bfloat16), wq_bf[...],
                         preferred_element_type=jnp.float32)
        q_all[...] = (q_full * SCALE).astype(jnp.bfloat16)

        def compute_chunk(c):
            rows = chunk_rows(c)
            qb = q_all[rows, :]
            groups = []
            for g in range(2):
                qg = qb[:, g * 4 * DH:(g + 1) * 4 * DH]
                qs = qg.reshape(CH, 4, DH).transpose(1, 0, 2)
                qs = qs.reshape(4 * CH, DH)
                s = lax.dot_general(
                    qs, kv_bf[0, g], (((1,), (1,)), ((), ())),
                    preferred_element_type=jnp.float32)
                p = jnp.exp(s)
                l = jnp.sum(p, axis=1, keepdims=True)
                og = jnp.dot(p.astype(jnp.bfloat16), kv_bf[1, g],
                             preferred_element_type=jnp.float32) * (1.0 / l)
                og = og.reshape(4, CH, DH).transpose(1, 0, 2)
                groups.append(og.reshape(CH, 4 * DH).astype(jnp.bfloat16))
            attn = jnp.concatenate(groups, axis=1)
            out_ref[0, rows, :] = jnp.dot(
                attn, wo_bf[...], preferred_element_type=jnp.float32)

        def send(src, dst, ssem, rsem, dev):
            r = pltpu.make_async_remote_copy(
                src_ref=src, dst_ref=dst, send_sem=ssem, recv_sem=rsem,
                device_id=(dev,), device_id_type=pl.DeviceIdType.MESH)
            r.start()
            return r

        compute_chunk(my + 4)
        compute_chunk(my - 3)
        for s in range(4):
            rstage[s] = (
                out_ref[0, chunk_rows(my + 4 - s), :] if s == 0
                else out_ref[0, chunk_rows(my + 4 - s), :]
                + rland[s - 1].astype(jnp.float32)
            ).astype(jnp.bfloat16)
            r_rdma = send(rstage.at[s], rland.at[s],
                          r_send_sems.at[s], r_recv_sems.at[s], right)
            l_rdma = None
            if s < 3:
                lstage[s] = (
                    out_ref[0, chunk_rows(my - 3 + s), :] if s == 0
                    else out_ref[0, chunk_rows(my - 3 + s), :]
                    + lland[s - 1].astype(jnp.float32)
                ).astype(jnp.bfloat16)
                l_rdma = send(lstage.at[s], lland.at[s],
                              l_send_sems.at[s], l_recv_sems.at[s], left)
            if s == 0:
                compute_chunk(my + 3)
                compute_chunk(my - 2)
            elif s == 1:
                compute_chunk(my + 2)
                compute_chunk(my - 1)
            elif s == 2:
                compute_chunk(my + 1)
                compute_chunk(my)
            if l_rdma is not None:
                l_rdma.wait()
            r_rdma.wait()

        out_ref[0, chunk_rows(my), :] += (
            rland[3].astype(jnp.float32) + lland[2].astype(jnp.float32))

        ag_own[...] = out_ref[0, chunk_rows(my), :].astype(jnp.bfloat16)
        rdmas = []
        for k in range(1, N_DEV):
            rdmas.append(send(ag_own, ag_land.at[k - 1],
                              ag_send_sems.at[k - 1], ag_recv_sems.at[k - 1],
                              lax.rem(my + k, N_DEV)))
        for k in range(1, N_DEV):
            rdmas[k - 1].wait()
            out_ref[0, chunk_rows(my - k), :] = (
                ag_land[k - 1].astype(jnp.float32))

    return pl.pallas_call(
        body,
        out_shape=jax.ShapeDtypeStruct((1, SQ, D), jnp.float32),
        in_specs=[
            pl.BlockSpec(memory_space=pltpu.MemorySpace.VMEM),
            pl.BlockSpec(memory_space=pltpu.MemorySpace.VMEM),
            pl.BlockSpec(memory_space=pltpu.MemorySpace.VMEM),
            pl.BlockSpec(memory_space=pltpu.MemorySpace.HBM),
            pl.BlockSpec(memory_space=pltpu.MemorySpace.HBM),
        ],
        out_specs=pl.BlockSpec(memory_space=pltpu.MemorySpace.VMEM),
        scratch_shapes=[
            pltpu.VMEM((2, 2, SKV, DH), jnp.float32),
            pltpu.VMEM((D, D), jnp.bfloat16),
            pltpu.VMEM((D, D), jnp.bfloat16),
            pltpu.VMEM((2, 2, SKV, DH), jnp.bfloat16),
            pltpu.VMEM((SQ, D), jnp.bfloat16),
            pltpu.VMEM((4, CH, D), jnp.bfloat16),
            pltpu.VMEM((4, CH, D), jnp.bfloat16),
            pltpu.VMEM((3, CH, D), jnp.bfloat16),
            pltpu.VMEM((3, CH, D), jnp.bfloat16),
            pltpu.VMEM((CH, D), jnp.bfloat16),
            pltpu.VMEM((N_DEV - 1, CH, D), jnp.bfloat16),
            pltpu.SemaphoreType.DMA((4,)),
            pltpu.SemaphoreType.DMA((4,)),
            pltpu.SemaphoreType.DMA((4,)),
            pltpu.SemaphoreType.DMA((3,)),
            pltpu.SemaphoreType.DMA((3,)),
            pltpu.SemaphoreType.DMA((N_DEV - 1,)),
            pltpu.SemaphoreType.DMA((N_DEV - 1,)),
        ],
        compiler_params=pltpu.CompilerParams(collective_id=0),
    )(x, Wq, Wo, K_ext, V_ext)
